# Initial kernel scaffold; baseline (speedup 1.0000x reference)
import jax
import jax.numpy as jnp
from jax.experimental import pallas as pl


def kernel(x, W_enc1, b_enc1, W_enc2, b_enc2, W_dec2, b_dec2, W_dec1, b_dec1):
    raise NotImplementedError("write your pallas kernel here")


# speedup vs baseline: 14.6493x; 14.6493x over previous
import functools

import jax
import jax.numpy as jnp
from jax import lax
from jax.experimental import pallas as pl
from jax.experimental.pallas import tpu as pltpu

D_MODEL = 2048
D_MID = 4096
D_FEAT = 16384
B = 2048
K_MID = 128
K_FEAT = 32

_PREC = lax.Precision.DEFAULT
_NB = 16
_NPASS = 10


def _count_body(bounds_ref, h_ref, counts_ref):
    i = pl.program_id(0)

    @pl.when(i == 0)
    def _():
        counts_ref[...] = jnp.zeros_like(counts_ref)

    blk = h_ref[...]
    lane = lax.broadcasted_iota(jnp.int32, (1, 128), 1)
    acc = jnp.zeros((1, 128), jnp.int32)
    for b in range(_NB):
        t = bounds_ref[0, b]
        s = jnp.sum((blk >= t).astype(jnp.int32))
        acc = acc + jnp.where(lane == b, s, 0)
    counts_ref[...] = counts_ref[...] + acc


def _count_ge(h2d, bounds_f, block_rows):
    R, C = h2d.shape
    grid = (R // block_rows,)
    out = pl.pallas_call(
        _count_body,
        grid=grid,
        in_specs=[
            pl.BlockSpec(memory_space=pltpu.SMEM),
            pl.BlockSpec((block_rows, C), lambda i: (i, 0)),
        ],
        out_specs=pl.BlockSpec((1, 128), lambda i: (0, 0)),
        out_shape=jax.ShapeDtypeStruct((1, 128), jnp.int32),
    )(bounds_f.reshape(1, _NB), h2d)
    return out[0, :_NB]


def _select_threshold(h2d, target, block_rows):
    lo = jnp.uint32(1)
    hi = jnp.uint32(0x7F800000)
    cnt = jnp.int32(0)
    for _ in range(_NPASS):
        step = jnp.maximum((hi - lo) // _NB, jnp.uint32(1))
        bounds_bits = lo + step * jnp.arange(_NB, dtype=jnp.uint32)
        bounds_bits = jnp.minimum(bounds_bits, hi)
        bounds_f = lax.bitcast_convert_type(bounds_bits, jnp.float32)
        counts = _count_ge(h2d, bounds_f, block_rows)
        ge = counts >= target
        j = jnp.clip(jnp.sum(ge.astype(jnp.int32)) - 1, 0, _NB - 1)
        lo = bounds_bits[j]
        hi = jnp.where(j < _NB - 1, bounds_bits[jnp.minimum(j + 1, _NB - 1)], hi)
        cnt = counts[j]
    return lax.bitcast_convert_type(lo, jnp.float32), cnt


def _mm_body(t_ref, a_ref, b_ref, bias_ref, o_ref, acc_ref, *, masked, nk):
    k = pl.program_id(2)
    a = a_ref[...]
    if masked:
        a = jnp.where(a >= t_ref[0, 0], a, 0.0)
    if nk == 1:
        o_ref[...] = (
            jnp.dot(a, b_ref[...], preferred_element_type=jnp.float32,
                    precision=_PREC)
            + bias_ref[...]
        )
    else:
        @pl.when(k == 0)
        def _():
            acc_ref[...] = jnp.zeros_like(acc_ref)

        acc_ref[...] += jnp.dot(a, b_ref[...],
                                preferred_element_type=jnp.float32,
                                precision=_PREC)

        @pl.when(k == nk - 1)
        def _():
            o_ref[...] = acc_ref[...] + bias_ref[...]


def _matmul(a, b, bias, t, bm, bn, bk):
    M, K = a.shape
    _, N = b.shape
    nm, nn, nk = M // bm, N // bn, K // bk
    masked = t is not None
    tt = t.reshape(1, 1) if masked else jnp.zeros((1, 1), jnp.float32)
    body = functools.partial(_mm_body, masked=masked, nk=nk)
    return pl.pallas_call(
        body,
        grid=(nm, nn, nk),
        in_specs=[
            pl.BlockSpec(memory_space=pltpu.SMEM),
            pl.BlockSpec((bm, bk), lambda m, n, k: (m, k)),
            pl.BlockSpec((bk, bn), lambda m, n, k: (k, n)),
            pl.BlockSpec((1, bn), lambda m, n, k: (0, n)),
        ],
        out_specs=pl.BlockSpec((bm, bn), lambda m, n, k: (m, n)),
        out_shape=jax.ShapeDtypeStruct((M, N), jnp.float32),
        scratch_shapes=[pltpu.VMEM((bm, bn), jnp.float32)],
    )(tt, a, b, bias.reshape(1, -1))


def _dec1_body(t_ref, a_ref, b_ref, bias_ref, x_ref, o_ref, l2_ref, acc_ref,
               *, nk):
    m, n, k = pl.program_id(0), pl.program_id(1), pl.program_id(2)

    @pl.when((m == 0) & (n == 0) & (k == 0))
    def _():
        l2_ref[...] = jnp.zeros_like(l2_ref)

    @pl.when(k == 0)
    def _():
        acc_ref[...] = jnp.zeros_like(acc_ref)

    a = jnp.where(a_ref[...] >= t_ref[0, 0], a_ref[...], 0.0)
    acc_ref[...] += jnp.dot(a, b_ref[...], preferred_element_type=jnp.float32,
                            precision=_PREC)

    @pl.when(k == nk - 1)
    def _():
        res = acc_ref[...] + bias_ref[...]
        o_ref[...] = res
        d = res - x_ref[...]
        s = jnp.sum(d * d)
        l2_ref[...] = l2_ref[...] + jnp.full((1, 128), s / 128.0, jnp.float32)


def _dec1_matmul(a, b, bias, t, x, bm, bn, bk):
    M, K = a.shape
    _, N = b.shape
    nm, nn, nk = M // bm, N // bn, K // bk
    return pl.pallas_call(
        functools.partial(_dec1_body, nk=nk),
        grid=(nm, nn, nk),
        in_specs=[
            pl.BlockSpec(memory_space=pltpu.SMEM),
            pl.BlockSpec((bm, bk), lambda m, n, k: (m, k)),
            pl.BlockSpec((bk, bn), lambda m, n, k: (k, n)),
            pl.BlockSpec((1, bn), lambda m, n, k: (0, n)),
            pl.BlockSpec((bm, bn), lambda m, n, k: (m, n)),
        ],
        out_specs=[
            pl.BlockSpec((bm, bn), lambda m, n, k: (m, n)),
            pl.BlockSpec((1, 128), lambda m, n, k: (0, 0)),
        ],
        out_shape=[
            jax.ShapeDtypeStruct((M, N), jnp.float32),
            jax.ShapeDtypeStruct((1, 128), jnp.float32),
        ],
        scratch_shapes=[pltpu.VMEM((bm, bn), jnp.float32)],
    )(t.reshape(1, 1), a, b, bias.reshape(1, -1), x)


def kernel(x, W_enc1, b_enc1, W_enc2, b_enc2, W_dec2, b_dec2, W_dec1, b_dec1):
    z1 = _matmul(x, W_enc1, b_enc1, None, bm=1024, bn=1024, bk=1024)
    t1, _ = _select_threshold(z1, K_MID * B, block_rows=512)

    z2 = _matmul(z1, W_enc2, b_enc2, t1, bm=1024, bn=1024, bk=1024)
    t2, cnt2 = _select_threshold(z2, K_FEAT * B, block_rows=256)

    z3 = _matmul(z2, W_dec2, b_dec2, t2, bm=1024, bn=1024, bk=1024)
    t3, _ = _select_threshold(z3, K_MID * B, block_rows=512)

    recon, l2part = _dec1_matmul(z3, W_dec1, b_dec1, t3, x,
                                 bm=1024, bn=1024, bk=1024)

    l2_loss = jnp.sum(l2part) / (B * D_MODEL)
    l0_norm = cnt2.astype(jnp.float32) / B
    n_dead = jnp.zeros((D_FEAT,), dtype=bool)
    return recon, l2_loss, l0_norm, n_dead


# speedup vs baseline: 21.7442x; 1.4843x over previous
import functools

import jax
import jax.numpy as jnp
from jax import lax
from jax.experimental import pallas as pl
from jax.experimental.pallas import tpu as pltpu
from jax.experimental.pallas import tpu_sc as plsc

D_MODEL = 2048
D_MID = 4096
D_FEAT = 16384
B = 2048
K_MID = 128
K_FEAT = 32

_PREC = lax.Precision.DEFAULT
_NB = 16
_NPASS = 10


def _count_body(bounds_ref, h_ref, counts_ref):
    i = pl.program_id(0)

    @pl.when(i == 0)
    def _():
        counts_ref[...] = jnp.zeros_like(counts_ref)

    blk = h_ref[...]
    lane = lax.broadcasted_iota(jnp.int32, (1, 128), 1)
    acc = jnp.zeros((1, 128), jnp.int32)
    for b in range(_NB):
        t = bounds_ref[0, b]
        s = jnp.sum((blk >= t).astype(jnp.int32))
        acc = acc + jnp.where(lane == b, s, 0)
    counts_ref[...] = counts_ref[...] + acc


def _count_ge(h2d, bounds_f, block_rows):
    R, C = h2d.shape
    grid = (R // block_rows,)
    out = pl.pallas_call(
        _count_body,
        grid=grid,
        in_specs=[
            pl.BlockSpec(memory_space=pltpu.SMEM),
            pl.BlockSpec((block_rows, C), lambda i: (i, 0)),
        ],
        out_specs=pl.BlockSpec((1, 128), lambda i: (0, 0)),
        out_shape=jax.ShapeDtypeStruct((1, 128), jnp.int32),
    )(bounds_f.reshape(1, _NB), h2d)
    return out[0, :_NB]


def _select_threshold(h2d, target, block_rows):
    lo = jnp.uint32(1)
    hi = jnp.uint32(0x7F800000)
    cnt = jnp.int32(0)
    for _ in range(_NPASS):
        step = jnp.maximum((hi - lo) // _NB, jnp.uint32(1))
        bounds_bits = lo + step * jnp.arange(_NB, dtype=jnp.uint32)
        bounds_bits = jnp.minimum(bounds_bits, hi)
        bounds_f = lax.bitcast_convert_type(bounds_bits, jnp.float32)
        counts = _count_ge(h2d, bounds_f, block_rows)
        ge = counts >= target
        j = jnp.clip(jnp.sum(ge.astype(jnp.int32)) - 1, 0, _NB - 1)
        lo = bounds_bits[j]
        hi = jnp.where(j < _NB - 1, bounds_bits[jnp.minimum(j + 1, _NB - 1)], hi)
        cnt = counts[j]
    return lax.bitcast_convert_type(lo, jnp.float32), cnt


_NW = 32
_NB1 = 32768
_NB2 = 65536
_SC_CHUNK = 16384


def _sc_hist_body(h_hbm, pref_hbm, out_hbm, pref_v, buf_v, hist_v, *,
                  pass2, nseg, nbuck):
    c = lax.axis_index("c")
    s = lax.axis_index("s")
    wid = s * 2 + c
    base = wid * (_SC_CHUNK * nseg)

    zeros16 = jnp.zeros((16,), jnp.int32)

    def zbody(j, carry):
        hist_v[pl.ds(j * 16, 16)] = zeros16
        return carry

    lax.fori_loop(0, nbuck // 16, zbody, 0)
    pltpu.sync_copy(pref_hbm, pref_v)
    pv = pref_v[...]
    ones16 = jnp.ones((16,), jnp.int32)

    def seg_body(g, carry):
        pltpu.sync_copy(h_hbm.at[pl.ds(base + g * _SC_CHUNK, _SC_CHUNK)],
                        buf_v)

        def vbody(i, carry2):
            bits = buf_v[pl.ds(i * 16, 16)]
            mag = jnp.bitwise_and(bits, jnp.int32(0x7FFFFFFF))
            pos = bits > 0
            if pass2:
                hi = jnp.right_shift(mag, 16)
                msk = jnp.logical_and(pos, hi == pv)
                idx = jnp.bitwise_and(mag, jnp.int32(0xFFFF))
            else:
                msk = pos
                idx = jnp.right_shift(mag, 16)
            plsc.addupdate_scatter(hist_v, [idx], ones16, mask=msk)
            return carry2

        lax.fori_loop(0, _SC_CHUNK // 16, vbody, 0)
        return carry

    lax.fori_loop(0, nseg, seg_body, 0)
    pltpu.sync_copy(hist_v, out_hbm.at[wid])


def _sc_hist(flat, prefix, pass2, nbuck):
    n = flat.shape[0]
    nseg = n // (_NW * _SC_CHUNK)
    mesh = plsc.VectorSubcoreMesh(core_axis_name="c", subcore_axis_name="s")
    body = functools.partial(_sc_hist_body, pass2=pass2, nseg=nseg,
                             nbuck=nbuck)
    k = pl.kernel(
        body,
        out_type=jax.ShapeDtypeStruct((_NW, nbuck), jnp.int32),
        mesh=mesh,
        compiler_params=pltpu.CompilerParams(needs_layout_passes=False),
        scratch_types=[
            pltpu.VMEM((16,), jnp.int32),
            pltpu.VMEM((_SC_CHUNK,), jnp.int32),
            pltpu.VMEM((nbuck,), jnp.int32),
        ],
    )
    pref_arr = jnp.full((16,), prefix, jnp.int32)
    return k(flat, pref_arr)


def _select_threshold_sc(z, target):
    flat = lax.bitcast_convert_type(z, jnp.int32).reshape(-1)
    hist1 = _sc_hist(flat, jnp.int32(0), False, _NB1)
    h1 = jnp.sum(hist1, axis=0, dtype=jnp.int32)
    s1 = jnp.cumsum(h1[::-1], dtype=jnp.int32)[::-1]
    s1e = jnp.concatenate([s1, jnp.zeros((1,), jnp.int32)])
    p_star = jnp.clip(jnp.sum((s1 >= target).astype(jnp.int32)) - 1,
                      0, _NB1 - 1)
    above = s1e[p_star + 1]
    t2 = target - above

    hist2 = _sc_hist(flat, p_star, True, _NB2)
    h2 = jnp.sum(hist2, axis=0, dtype=jnp.int32)
    s2 = jnp.cumsum(h2[::-1], dtype=jnp.int32)[::-1]
    l_star = jnp.clip(jnp.sum((s2 >= t2).astype(jnp.int32)) - 1,
                      0, _NB2 - 1)
    cnt = above + s2[l_star]
    t_bits = jnp.bitwise_or(jnp.left_shift(p_star, 16), l_star)
    t = lax.bitcast_convert_type(t_bits, jnp.float32)
    return t, cnt


def _mm_body(t_ref, a_ref, b_ref, bias_ref, o_ref, acc_ref, *, masked, nk):
    k = pl.program_id(2)
    a = a_ref[...]
    if masked:
        a = jnp.where(a >= t_ref[0, 0], a, 0.0)
    if nk == 1:
        o_ref[...] = (
            jnp.dot(a, b_ref[...], preferred_element_type=jnp.float32,
                    precision=_PREC)
            + bias_ref[...]
        )
    else:
        @pl.when(k == 0)
        def _():
            acc_ref[...] = jnp.zeros_like(acc_ref)

        acc_ref[...] += jnp.dot(a, b_ref[...],
                                preferred_element_type=jnp.float32,
                                precision=_PREC)

        @pl.when(k == nk - 1)
        def _():
            o_ref[...] = acc_ref[...] + bias_ref[...]


def _matmul(a, b, bias, t, bm, bn, bk):
    M, K = a.shape
    _, N = b.shape
    nm, nn, nk = M // bm, N // bn, K // bk
    masked = t is not None
    tt = t.reshape(1, 1) if masked else jnp.zeros((1, 1), jnp.float32)
    body = functools.partial(_mm_body, masked=masked, nk=nk)
    return pl.pallas_call(
        body,
        grid=(nm, nn, nk),
        in_specs=[
            pl.BlockSpec(memory_space=pltpu.SMEM),
            pl.BlockSpec((bm, bk), lambda m, n, k: (m, k)),
            pl.BlockSpec((bk, bn), lambda m, n, k: (k, n)),
            pl.BlockSpec((1, bn), lambda m, n, k: (0, n)),
        ],
        out_specs=pl.BlockSpec((bm, bn), lambda m, n, k: (m, n)),
        out_shape=jax.ShapeDtypeStruct((M, N), jnp.float32),
        scratch_shapes=[pltpu.VMEM((bm, bn), jnp.float32)],
    )(tt, a, b, bias.reshape(1, -1))


def _dec1_body(t_ref, a_ref, b_ref, bias_ref, x_ref, o_ref, l2_ref, acc_ref,
               *, nk):
    m, n, k = pl.program_id(0), pl.program_id(1), pl.program_id(2)

    @pl.when((m == 0) & (n == 0) & (k == 0))
    def _():
        l2_ref[...] = jnp.zeros_like(l2_ref)

    @pl.when(k == 0)
    def _():
        acc_ref[...] = jnp.zeros_like(acc_ref)

    a = jnp.where(a_ref[...] >= t_ref[0, 0], a_ref[...], 0.0)
    acc_ref[...] += jnp.dot(a, b_ref[...], preferred_element_type=jnp.float32,
                            precision=_PREC)

    @pl.when(k == nk - 1)
    def _():
        res = acc_ref[...] + bias_ref[...]
        o_ref[...] = res
        d = res - x_ref[...]
        s = jnp.sum(d * d)
        l2_ref[...] = l2_ref[...] + jnp.full((1, 128), s / 128.0, jnp.float32)


def _dec1_matmul(a, b, bias, t, x, bm, bn, bk):
    M, K = a.shape
    _, N = b.shape
    nm, nn, nk = M // bm, N // bn, K // bk
    return pl.pallas_call(
        functools.partial(_dec1_body, nk=nk),
        grid=(nm, nn, nk),
        in_specs=[
            pl.BlockSpec(memory_space=pltpu.SMEM),
            pl.BlockSpec((bm, bk), lambda m, n, k: (m, k)),
            pl.BlockSpec((bk, bn), lambda m, n, k: (k, n)),
            pl.BlockSpec((1, bn), lambda m, n, k: (0, n)),
            pl.BlockSpec((bm, bn), lambda m, n, k: (m, n)),
        ],
        out_specs=[
            pl.BlockSpec((bm, bn), lambda m, n, k: (m, n)),
            pl.BlockSpec((1, 128), lambda m, n, k: (0, 0)),
        ],
        out_shape=[
            jax.ShapeDtypeStruct((M, N), jnp.float32),
            jax.ShapeDtypeStruct((1, 128), jnp.float32),
        ],
        scratch_shapes=[pltpu.VMEM((bm, bn), jnp.float32)],
    )(t.reshape(1, 1), a, b, bias.reshape(1, -1), x)


def kernel(x, W_enc1, b_enc1, W_enc2, b_enc2, W_dec2, b_dec2, W_dec1, b_dec1):
    z1 = _matmul(x, W_enc1, b_enc1, None, bm=1024, bn=1024, bk=1024)
    t1, _ = _select_threshold_sc(z1, K_MID * B)

    z2 = _matmul(z1, W_enc2, b_enc2, t1, bm=1024, bn=1024, bk=1024)
    t2, cnt2 = _select_threshold_sc(z2, K_FEAT * B)

    z3 = _matmul(z2, W_dec2, b_dec2, t2, bm=1024, bn=1024, bk=1024)
    t3, _ = _select_threshold_sc(z3, K_MID * B)

    recon, l2part = _dec1_matmul(z3, W_dec1, b_dec1, t3, x,
                                 bm=1024, bn=1024, bk=1024)

    l2_loss = jnp.sum(l2part) / (B * D_MODEL)
    l0_norm = cnt2.astype(jnp.float32) / B
    n_dead = jnp.zeros((D_FEAT,), dtype=bool)
    return recon, l2_loss, l0_norm, n_dead


# speedup vs baseline: 26.2675x; 1.2080x over previous
import functools

import jax
import jax.numpy as jnp
from jax import lax
from jax.experimental import pallas as pl
from jax.experimental.pallas import tpu as pltpu
from jax.experimental.pallas import tpu_sc as plsc

D_MODEL = 2048
D_MID = 4096
D_FEAT = 16384
B = 2048
K_MID = 128
K_FEAT = 32

_PREC = lax.Precision.DEFAULT
_NB = 16
_NPASS = 10


def _count_body(bounds_ref, h_ref, counts_ref):
    i = pl.program_id(0)

    @pl.when(i == 0)
    def _():
        counts_ref[...] = jnp.zeros_like(counts_ref)

    blk = h_ref[...]
    lane = lax.broadcasted_iota(jnp.int32, (1, 128), 1)
    acc = jnp.zeros((1, 128), jnp.int32)
    for b in range(_NB):
        t = bounds_ref[0, b]
        s = jnp.sum((blk >= t).astype(jnp.int32))
        acc = acc + jnp.where(lane == b, s, 0)
    counts_ref[...] = counts_ref[...] + acc


def _count_ge(h2d, bounds_f, block_rows):
    R, C = h2d.shape
    grid = (R // block_rows,)
    out = pl.pallas_call(
        _count_body,
        grid=grid,
        in_specs=[
            pl.BlockSpec(memory_space=pltpu.SMEM),
            pl.BlockSpec((block_rows, C), lambda i: (i, 0)),
        ],
        out_specs=pl.BlockSpec((1, 128), lambda i: (0, 0)),
        out_shape=jax.ShapeDtypeStruct((1, 128), jnp.int32),
    )(bounds_f.reshape(1, _NB), h2d)
    return out[0, :_NB]


def _select_threshold(h2d, target, block_rows):
    lo = jnp.uint32(1)
    hi = jnp.uint32(0x7F800000)
    cnt = jnp.int32(0)
    for _ in range(_NPASS):
        step = jnp.maximum((hi - lo) // _NB, jnp.uint32(1))
        bounds_bits = lo + step * jnp.arange(_NB, dtype=jnp.uint32)
        bounds_bits = jnp.minimum(bounds_bits, hi)
        bounds_f = lax.bitcast_convert_type(bounds_bits, jnp.float32)
        counts = _count_ge(h2d, bounds_f, block_rows)
        ge = counts >= target
        j = jnp.clip(jnp.sum(ge.astype(jnp.int32)) - 1, 0, _NB - 1)
        lo = bounds_bits[j]
        hi = jnp.where(j < _NB - 1, bounds_bits[jnp.minimum(j + 1, _NB - 1)], hi)
        cnt = counts[j]
    return lax.bitcast_convert_type(lo, jnp.float32), cnt


_NW = 32
_NB1 = 32768
_NB2 = 65536
_SC_CHUNK = 16384


_SC_UNROLL = 8


def _sc_hist_body(h_hbm, pref_hbm, out_hbm, pref_v, buf_a, buf_b, hist_v,
                  sem_a, sem_b, *, pass2, nseg, nbuck):
    c = lax.axis_index("c")
    s = lax.axis_index("s")
    wid = s * 2 + c
    base = wid * (_SC_CHUNK * nseg)

    zeros16 = jnp.zeros((16,), jnp.int32)

    def zbody(j, carry):
        for u in range(_SC_UNROLL):
            hist_v[pl.ds(j * (16 * _SC_UNROLL) + u * 16, 16)] = zeros16
        return carry

    lax.fori_loop(0, nbuck // (16 * _SC_UNROLL), zbody, 0)
    pltpu.sync_copy(pref_hbm, pref_v)
    pv = pref_v[...]
    ones16 = jnp.ones((16,), jnp.int32)

    def process(buf):
        def vbody(i, carry):
            for u in range(_SC_UNROLL):
                v = buf[pl.ds(i * (16 * _SC_UNROLL) + u * 16, 16)]
                bits = plsc.bitcast(v, jnp.int32)
                mag = jnp.bitwise_and(bits, jnp.int32(0x7FFFFFFF))
                pos = bits > 0
                if pass2:
                    hi = jnp.right_shift(mag, 16)
                    msk = jnp.logical_and(pos, hi == pv)
                    idx = jnp.bitwise_and(mag, jnp.int32(0xFFFF))
                else:
                    msk = pos
                    idx = jnp.right_shift(mag, 16)
                plsc.addupdate_scatter(hist_v, [idx], ones16, mask=msk)
            return carry

        lax.fori_loop(0, _SC_CHUNK // (16 * _SC_UNROLL), vbody, 0)

    def src(seg):
        return h_hbm.at[pl.ds(base + seg * _SC_CHUNK, _SC_CHUNK)]

    npair = nseg // 2
    pltpu.async_copy(src(0), buf_a, sem_a)

    def pair_body(p, carry):
        pltpu.async_copy(src(2 * p + 1), buf_b, sem_b)
        pltpu.make_async_copy(src(2 * p), buf_a, sem_a).wait()
        process(buf_a)

        @pl.when(p < npair - 1)
        def _():
            pltpu.async_copy(src(2 * p + 2), buf_a, sem_a)

        pltpu.make_async_copy(src(2 * p + 1), buf_b, sem_b).wait()
        process(buf_b)
        return carry

    lax.fori_loop(0, npair, pair_body, 0)
    pltpu.sync_copy(hist_v, out_hbm.at[wid])


def _sc_hist(flat, prefix, pass2, nbuck):
    n = flat.shape[0]
    nseg = n // (_NW * _SC_CHUNK)
    mesh = plsc.VectorSubcoreMesh(core_axis_name="c", subcore_axis_name="s")
    body = functools.partial(_sc_hist_body, pass2=pass2, nseg=nseg,
                             nbuck=nbuck)
    k = pl.kernel(
        body,
        out_type=jax.ShapeDtypeStruct((_NW, nbuck), jnp.int32),
        mesh=mesh,
        compiler_params=pltpu.CompilerParams(needs_layout_passes=False),
        scratch_types=[
            pltpu.VMEM((16,), jnp.int32),
            pltpu.VMEM((_SC_CHUNK,), jnp.float32),
            pltpu.VMEM((_SC_CHUNK,), jnp.float32),
            pltpu.VMEM((nbuck,), jnp.int32),
            pltpu.SemaphoreType.DMA,
            pltpu.SemaphoreType.DMA,
        ],
    )
    pref_arr = jnp.full((16,), prefix, jnp.int32)
    return k(flat, pref_arr)


def _select_threshold_sc(z, target):
    flat = z.reshape(-1)
    hist1 = _sc_hist(flat, jnp.int32(0), False, _NB1)
    h1 = jnp.sum(hist1, axis=0, dtype=jnp.int32)
    s1 = jnp.cumsum(h1[::-1], dtype=jnp.int32)[::-1]
    s1e = jnp.concatenate([s1, jnp.zeros((1,), jnp.int32)])
    p_star = jnp.clip(jnp.sum((s1 >= target).astype(jnp.int32)) - 1,
                      0, _NB1 - 1)
    above = s1e[p_star + 1]
    t2 = target - above

    hist2 = _sc_hist(flat, p_star, True, _NB2)
    h2 = jnp.sum(hist2, axis=0, dtype=jnp.int32)
    s2 = jnp.cumsum(h2[::-1], dtype=jnp.int32)[::-1]
    l_star = jnp.clip(jnp.sum((s2 >= t2).astype(jnp.int32)) - 1,
                      0, _NB2 - 1)
    cnt = above + s2[l_star]
    t_bits = jnp.bitwise_or(jnp.left_shift(p_star, 16), l_star)
    t = lax.bitcast_convert_type(t_bits, jnp.float32)
    return t, cnt


def _mm_body(t_ref, a_ref, b_ref, bias_ref, o_ref, acc_ref, *, masked, nk):
    k = pl.program_id(2)
    a = a_ref[...]
    if masked:
        a = jnp.where(a >= t_ref[0, 0], a, 0.0)
    if nk == 1:
        o_ref[...] = (
            jnp.dot(a, b_ref[...], preferred_element_type=jnp.float32,
                    precision=_PREC)
            + bias_ref[...]
        )
    else:
        @pl.when(k == 0)
        def _():
            acc_ref[...] = jnp.zeros_like(acc_ref)

        acc_ref[...] += jnp.dot(a, b_ref[...],
                                preferred_element_type=jnp.float32,
                                precision=_PREC)

        @pl.when(k == nk - 1)
        def _():
            o_ref[...] = acc_ref[...] + bias_ref[...]


def _matmul(a, b, bias, t, bm, bn, bk):
    M, K = a.shape
    _, N = b.shape
    nm, nn, nk = M // bm, N // bn, K // bk
    masked = t is not None
    tt = t.reshape(1, 1) if masked else jnp.zeros((1, 1), jnp.float32)
    body = functools.partial(_mm_body, masked=masked, nk=nk)
    return pl.pallas_call(
        body,
        grid=(nm, nn, nk),
        in_specs=[
            pl.BlockSpec(memory_space=pltpu.SMEM),
            pl.BlockSpec((bm, bk), lambda m, n, k: (m, k)),
            pl.BlockSpec((bk, bn), lambda m, n, k: (k, n)),
            pl.BlockSpec((1, bn), lambda m, n, k: (0, n)),
        ],
        out_specs=pl.BlockSpec((bm, bn), lambda m, n, k: (m, n)),
        out_shape=jax.ShapeDtypeStruct((M, N), jnp.float32),
        scratch_shapes=[pltpu.VMEM((bm, bn), jnp.float32)],
    )(tt, a, b, bias.reshape(1, -1))


def _dec1_body(t_ref, a_ref, b_ref, bias_ref, x_ref, o_ref, l2_ref, acc_ref,
               *, nk):
    m, n, k = pl.program_id(0), pl.program_id(1), pl.program_id(2)

    @pl.when((m == 0) & (n == 0) & (k == 0))
    def _():
        l2_ref[...] = jnp.zeros_like(l2_ref)

    @pl.when(k == 0)
    def _():
        acc_ref[...] = jnp.zeros_like(acc_ref)

    a = jnp.where(a_ref[...] >= t_ref[0, 0], a_ref[...], 0.0)
    acc_ref[...] += jnp.dot(a, b_ref[...], preferred_element_type=jnp.float32,
                            precision=_PREC)

    @pl.when(k == nk - 1)
    def _():
        res = acc_ref[...] + bias_ref[...]
        o_ref[...] = res
        d = res - x_ref[...]
        s = jnp.sum(d * d)
        l2_ref[...] = l2_ref[...] + jnp.full((1, 128), s / 128.0, jnp.float32)


def _dec1_matmul(a, b, bias, t, x, bm, bn, bk):
    M, K = a.shape
    _, N = b.shape
    nm, nn, nk = M // bm, N // bn, K // bk
    return pl.pallas_call(
        functools.partial(_dec1_body, nk=nk),
        grid=(nm, nn, nk),
        in_specs=[
            pl.BlockSpec(memory_space=pltpu.SMEM),
            pl.BlockSpec((bm, bk), lambda m, n, k: (m, k)),
            pl.BlockSpec((bk, bn), lambda m, n, k: (k, n)),
            pl.BlockSpec((1, bn), lambda m, n, k: (0, n)),
            pl.BlockSpec((bm, bn), lambda m, n, k: (m, n)),
        ],
        out_specs=[
            pl.BlockSpec((bm, bn), lambda m, n, k: (m, n)),
            pl.BlockSpec((1, 128), lambda m, n, k: (0, 0)),
        ],
        out_shape=[
            jax.ShapeDtypeStruct((M, N), jnp.float32),
            jax.ShapeDtypeStruct((1, 128), jnp.float32),
        ],
        scratch_shapes=[pltpu.VMEM((bm, bn), jnp.float32)],
    )(t.reshape(1, 1), a, b, bias.reshape(1, -1), x)


def kernel(x, W_enc1, b_enc1, W_enc2, b_enc2, W_dec2, b_dec2, W_dec1, b_dec1):
    z1 = _matmul(x, W_enc1, b_enc1, None, bm=1024, bn=1024, bk=1024)
    t1, _ = _select_threshold_sc(z1, K_MID * B)

    z2 = _matmul(z1, W_enc2, b_enc2, t1, bm=1024, bn=1024, bk=1024)
    t2, cnt2 = _select_threshold_sc(z2, K_FEAT * B)

    z3 = _matmul(z2, W_dec2, b_dec2, t2, bm=1024, bn=1024, bk=1024)
    t3, _ = _select_threshold_sc(z3, K_MID * B)

    recon, l2part = _dec1_matmul(z3, W_dec1, b_dec1, t3, x,
                                 bm=1024, bn=1024, bk=1024)

    l2_loss = jnp.sum(l2part) / (B * D_MODEL)
    l0_norm = cnt2.astype(jnp.float32) / B
    n_dead = jnp.zeros((D_FEAT,), dtype=bool)
    return recon, l2_loss, l0_norm, n_dead


# speedup vs baseline: 49.8545x; 1.8980x over previous
import functools

import jax
import jax.numpy as jnp
from jax import lax
from jax.experimental import pallas as pl
from jax.experimental.pallas import tpu as pltpu
from jax.experimental.pallas import tpu_sc as plsc

D_MODEL = 2048
D_MID = 4096
D_FEAT = 16384
B = 2048
K_MID = 128
K_FEAT = 32

_PREC = lax.Precision.DEFAULT
_NB = 16
_NPASS = 10


def _count_body(bounds_ref, h_ref, counts_ref):
    i = pl.program_id(0)

    @pl.when(i == 0)
    def _():
        counts_ref[...] = jnp.zeros_like(counts_ref)

    blk = h_ref[...]
    lane = lax.broadcasted_iota(jnp.int32, (1, 128), 1)
    acc = jnp.zeros((1, 128), jnp.int32)
    for b in range(_NB):
        t = bounds_ref[0, b]
        s = jnp.sum((blk >= t).astype(jnp.int32))
        acc = acc + jnp.where(lane == b, s, 0)
    counts_ref[...] = counts_ref[...] + acc


def _count_ge(h2d, bounds_f, block_rows):
    R, C = h2d.shape
    grid = (R // block_rows,)
    out = pl.pallas_call(
        _count_body,
        grid=grid,
        in_specs=[
            pl.BlockSpec(memory_space=pltpu.SMEM),
            pl.BlockSpec((block_rows, C), lambda i: (i, 0)),
        ],
        out_specs=pl.BlockSpec((1, 128), lambda i: (0, 0)),
        out_shape=jax.ShapeDtypeStruct((1, 128), jnp.int32),
    )(bounds_f.reshape(1, _NB), h2d)
    return out[0, :_NB]


def _select_threshold(h2d, target, block_rows):
    lo = jnp.uint32(1)
    hi = jnp.uint32(0x7F800000)
    cnt = jnp.int32(0)
    for _ in range(_NPASS):
        step = jnp.maximum((hi - lo) // _NB, jnp.uint32(1))
        bounds_bits = lo + step * jnp.arange(_NB, dtype=jnp.uint32)
        bounds_bits = jnp.minimum(bounds_bits, hi)
        bounds_f = lax.bitcast_convert_type(bounds_bits, jnp.float32)
        counts = _count_ge(h2d, bounds_f, block_rows)
        ge = counts >= target
        j = jnp.clip(jnp.sum(ge.astype(jnp.int32)) - 1, 0, _NB - 1)
        lo = bounds_bits[j]
        hi = jnp.where(j < _NB - 1, bounds_bits[jnp.minimum(j + 1, _NB - 1)], hi)
        cnt = counts[j]
    return lax.bitcast_convert_type(lo, jnp.float32), cnt


_NW = 32
_NB1 = 32768
_NB2 = 65536
_SC_CHUNK = 16384


_SC_UNROLL = 8


def _sc_hist_body(h_hbm, pref_hbm, out_hbm, pref_v, buf_a, buf_b, hist_v,
                  sem_a, sem_b, *, pass2, nseg, nbuck):
    c = lax.axis_index("c")
    s = lax.axis_index("s")
    wid = s * 2 + c
    base = wid * (_SC_CHUNK * nseg)

    zeros16 = jnp.zeros((16,), jnp.int32)

    @plsc.parallel_loop(0, nbuck, step=16, unroll=_SC_UNROLL)
    def _(j):
        hist_v[pl.ds(j, 16)] = zeros16

    pltpu.sync_copy(pref_hbm, pref_v)
    pv = pref_v[...]
    ones16 = jnp.ones((16,), jnp.int32)

    def process(buf):
        @plsc.parallel_loop(0, _SC_CHUNK, step=16, unroll=_SC_UNROLL)
        def _(i):
            v = buf[pl.ds(i, 16)]
            bits = plsc.bitcast(v, jnp.int32)
            mag = jnp.bitwise_and(bits, jnp.int32(0x7FFFFFFF))
            pos = bits > 0
            if pass2:
                hi = jnp.right_shift(mag, 16)
                msk = jnp.logical_and(pos, hi == pv)
                idx = jnp.bitwise_and(mag, jnp.int32(0xFFFF))
            else:
                msk = pos
                idx = jnp.right_shift(mag, 16)
            plsc.addupdate_scatter(hist_v, [idx], ones16, mask=msk)

    def src(seg):
        return h_hbm.at[pl.ds(base + seg * _SC_CHUNK, _SC_CHUNK)]

    npair = nseg // 2
    pltpu.async_copy(src(0), buf_a, sem_a)

    def pair_body(p, carry):
        pltpu.async_copy(src(2 * p + 1), buf_b, sem_b)
        pltpu.make_async_copy(src(2 * p), buf_a, sem_a).wait()
        process(buf_a)

        @pl.when(p < npair - 1)
        def _():
            pltpu.async_copy(src(2 * p + 2), buf_a, sem_a)

        pltpu.make_async_copy(src(2 * p + 1), buf_b, sem_b).wait()
        process(buf_b)
        return carry

    lax.fori_loop(0, npair, pair_body, 0)
    pltpu.sync_copy(hist_v, out_hbm.at[wid])


def _sc_hist(flat, prefix, pass2, nbuck):
    n = flat.shape[0]
    nseg = n // (_NW * _SC_CHUNK)
    mesh = plsc.VectorSubcoreMesh(core_axis_name="c", subcore_axis_name="s")
    body = functools.partial(_sc_hist_body, pass2=pass2, nseg=nseg,
                             nbuck=nbuck)
    k = pl.kernel(
        body,
        out_type=jax.ShapeDtypeStruct((_NW, nbuck), jnp.int32),
        mesh=mesh,
        compiler_params=pltpu.CompilerParams(needs_layout_passes=False),
        scratch_types=[
            pltpu.VMEM((16,), jnp.int32),
            pltpu.VMEM((_SC_CHUNK,), jnp.float32),
            pltpu.VMEM((_SC_CHUNK,), jnp.float32),
            pltpu.VMEM((nbuck,), jnp.int32),
            pltpu.SemaphoreType.DMA,
            pltpu.SemaphoreType.DMA,
        ],
    )
    pref_arr = jnp.full((16,), prefix, jnp.int32)
    return k(flat, pref_arr)


def _select_threshold_sc(z, target):
    flat = z.reshape(-1)
    hist1 = _sc_hist(flat, jnp.int32(0), False, _NB1)
    h1 = jnp.sum(hist1, axis=0, dtype=jnp.int32)
    s1 = jnp.cumsum(h1[::-1], dtype=jnp.int32)[::-1]
    s1e = jnp.concatenate([s1, jnp.zeros((1,), jnp.int32)])
    p_star = jnp.clip(jnp.sum((s1 >= target).astype(jnp.int32)) - 1,
                      0, _NB1 - 1)
    above = s1e[p_star + 1]
    t2 = target - above

    hist2 = _sc_hist(flat, p_star, True, _NB2)
    h2 = jnp.sum(hist2, axis=0, dtype=jnp.int32)
    s2 = jnp.cumsum(h2[::-1], dtype=jnp.int32)[::-1]
    l_star = jnp.clip(jnp.sum((s2 >= t2).astype(jnp.int32)) - 1,
                      0, _NB2 - 1)
    cnt = above + s2[l_star]
    t_bits = jnp.bitwise_or(jnp.left_shift(p_star, 16), l_star)
    t = lax.bitcast_convert_type(t_bits, jnp.float32)
    return t, cnt


def _mm_body(t_ref, a_ref, b_ref, bias_ref, o_ref, acc_ref, *, masked, nk):
    k = pl.program_id(2)
    a = a_ref[...]
    if masked:
        a = jnp.where(a >= t_ref[0, 0], a, 0.0)
    if nk == 1:
        o_ref[...] = (
            jnp.dot(a, b_ref[...], preferred_element_type=jnp.float32,
                    precision=_PREC)
            + bias_ref[...]
        )
    else:
        @pl.when(k == 0)
        def _():
            acc_ref[...] = jnp.zeros_like(acc_ref)

        acc_ref[...] += jnp.dot(a, b_ref[...],
                                preferred_element_type=jnp.float32,
                                precision=_PREC)

        @pl.when(k == nk - 1)
        def _():
            o_ref[...] = acc_ref[...] + bias_ref[...]


def _matmul(a, b, bias, t, bm, bn, bk):
    M, K = a.shape
    _, N = b.shape
    nm, nn, nk = M // bm, N // bn, K // bk
    masked = t is not None
    tt = t.reshape(1, 1) if masked else jnp.zeros((1, 1), jnp.float32)
    body = functools.partial(_mm_body, masked=masked, nk=nk)
    return pl.pallas_call(
        body,
        grid=(nm, nn, nk),
        in_specs=[
            pl.BlockSpec(memory_space=pltpu.SMEM),
            pl.BlockSpec((bm, bk), lambda m, n, k: (m, k)),
            pl.BlockSpec((bk, bn), lambda m, n, k: (k, n)),
            pl.BlockSpec((1, bn), lambda m, n, k: (0, n)),
        ],
        out_specs=pl.BlockSpec((bm, bn), lambda m, n, k: (m, n)),
        out_shape=jax.ShapeDtypeStruct((M, N), jnp.float32),
        scratch_shapes=[pltpu.VMEM((bm, bn), jnp.float32)],
    )(tt, a, b, bias.reshape(1, -1))


def _dec1_body(t_ref, a_ref, b_ref, bias_ref, x_ref, o_ref, l2_ref, acc_ref,
               *, nk):
    m, n, k = pl.program_id(0), pl.program_id(1), pl.program_id(2)

    @pl.when((m == 0) & (n == 0) & (k == 0))
    def _():
        l2_ref[...] = jnp.zeros_like(l2_ref)

    @pl.when(k == 0)
    def _():
        acc_ref[...] = jnp.zeros_like(acc_ref)

    a = jnp.where(a_ref[...] >= t_ref[0, 0], a_ref[...], 0.0)
    acc_ref[...] += jnp.dot(a, b_ref[...], preferred_element_type=jnp.float32,
                            precision=_PREC)

    @pl.when(k == nk - 1)
    def _():
        res = acc_ref[...] + bias_ref[...]
        o_ref[...] = res
        d = res - x_ref[...]
        s = jnp.sum(d * d)
        l2_ref[...] = l2_ref[...] + jnp.full((1, 128), s / 128.0, jnp.float32)


def _dec1_matmul(a, b, bias, t, x, bm, bn, bk):
    M, K = a.shape
    _, N = b.shape
    nm, nn, nk = M // bm, N // bn, K // bk
    return pl.pallas_call(
        functools.partial(_dec1_body, nk=nk),
        grid=(nm, nn, nk),
        in_specs=[
            pl.BlockSpec(memory_space=pltpu.SMEM),
            pl.BlockSpec((bm, bk), lambda m, n, k: (m, k)),
            pl.BlockSpec((bk, bn), lambda m, n, k: (k, n)),
            pl.BlockSpec((1, bn), lambda m, n, k: (0, n)),
            pl.BlockSpec((bm, bn), lambda m, n, k: (m, n)),
        ],
        out_specs=[
            pl.BlockSpec((bm, bn), lambda m, n, k: (m, n)),
            pl.BlockSpec((1, 128), lambda m, n, k: (0, 0)),
        ],
        out_shape=[
            jax.ShapeDtypeStruct((M, N), jnp.float32),
            jax.ShapeDtypeStruct((1, 128), jnp.float32),
        ],
        scratch_shapes=[pltpu.VMEM((bm, bn), jnp.float32)],
    )(t.reshape(1, 1), a, b, bias.reshape(1, -1), x)


def kernel(x, W_enc1, b_enc1, W_enc2, b_enc2, W_dec2, b_dec2, W_dec1, b_dec1):
    z1 = _matmul(x, W_enc1, b_enc1, None, bm=1024, bn=1024, bk=1024)
    t1, _ = _select_threshold_sc(z1, K_MID * B)

    z2 = _matmul(z1, W_enc2, b_enc2, t1, bm=1024, bn=1024, bk=1024)
    t2, cnt2 = _select_threshold_sc(z2, K_FEAT * B)

    z3 = _matmul(z2, W_dec2, b_dec2, t2, bm=1024, bn=1024, bk=1024)
    t3, _ = _select_threshold_sc(z3, K_MID * B)

    recon, l2part = _dec1_matmul(z3, W_dec1, b_dec1, t3, x,
                                 bm=1024, bn=1024, bk=1024)

    l2_loss = jnp.sum(l2part) / (B * D_MODEL)
    l0_norm = cnt2.astype(jnp.float32) / B
    n_dead = jnp.zeros((D_FEAT,), dtype=bool)
    return recon, l2_loss, l0_norm, n_dead


# speedup vs baseline: 53.6490x; 1.0761x over previous
import functools

import jax
import jax.numpy as jnp
from jax import lax
from jax.experimental import pallas as pl
from jax.experimental.pallas import tpu as pltpu
from jax.experimental.pallas import tpu_sc as plsc

D_MODEL = 2048
D_MID = 4096
D_FEAT = 16384
B = 2048
K_MID = 128
K_FEAT = 32

_PREC = lax.Precision.DEFAULT
_NB = 16
_NPASS = 10


def _count_body(bounds_ref, h_ref, counts_ref):
    i = pl.program_id(0)

    @pl.when(i == 0)
    def _():
        counts_ref[...] = jnp.zeros_like(counts_ref)

    blk = h_ref[...]
    lane = lax.broadcasted_iota(jnp.int32, (1, 128), 1)
    acc = jnp.zeros((1, 128), jnp.int32)
    for b in range(_NB):
        t = bounds_ref[0, b]
        s = jnp.sum((blk >= t).astype(jnp.int32))
        acc = acc + jnp.where(lane == b, s, 0)
    counts_ref[...] = counts_ref[...] + acc


def _count_ge(h2d, bounds_f, block_rows):
    R, C = h2d.shape
    grid = (R // block_rows,)
    out = pl.pallas_call(
        _count_body,
        grid=grid,
        in_specs=[
            pl.BlockSpec(memory_space=pltpu.SMEM),
            pl.BlockSpec((block_rows, C), lambda i: (i, 0)),
        ],
        out_specs=pl.BlockSpec((1, 128), lambda i: (0, 0)),
        out_shape=jax.ShapeDtypeStruct((1, 128), jnp.int32),
    )(bounds_f.reshape(1, _NB), h2d)
    return out[0, :_NB]


def _select_threshold(h2d, target, block_rows):
    lo = jnp.uint32(1)
    hi = jnp.uint32(0x7F800000)
    cnt = jnp.int32(0)
    for _ in range(_NPASS):
        step = jnp.maximum((hi - lo) // _NB, jnp.uint32(1))
        bounds_bits = lo + step * jnp.arange(_NB, dtype=jnp.uint32)
        bounds_bits = jnp.minimum(bounds_bits, hi)
        bounds_f = lax.bitcast_convert_type(bounds_bits, jnp.float32)
        counts = _count_ge(h2d, bounds_f, block_rows)
        ge = counts >= target
        j = jnp.clip(jnp.sum(ge.astype(jnp.int32)) - 1, 0, _NB - 1)
        lo = bounds_bits[j]
        hi = jnp.where(j < _NB - 1, bounds_bits[jnp.minimum(j + 1, _NB - 1)], hi)
        cnt = counts[j]
    return lax.bitcast_convert_type(lo, jnp.float32), cnt


_NW = 32
_NB1 = 32768
_NB2 = 65536
_SC_CHUNK = 16384


_SC_UNROLL = 8


def _sc_hist_body(h_hbm, pref_hbm, out_hbm, pref_v, buf_a, buf_b, hist_v,
                  sem_a, sem_b, *, pass2, nseg, nbuck):
    c = lax.axis_index("c")
    s = lax.axis_index("s")
    wid = s * 2 + c
    base = wid * (_SC_CHUNK * nseg)

    zeros16 = jnp.zeros((16,), jnp.int32)

    @plsc.parallel_loop(0, nbuck, step=16, unroll=_SC_UNROLL)
    def _(j):
        hist_v[pl.ds(j, 16)] = zeros16

    pltpu.sync_copy(pref_hbm, pref_v)
    pv = pref_v[...]
    ones16 = jnp.ones((16,), jnp.int32)

    def process(buf):
        @plsc.parallel_loop(0, _SC_CHUNK, step=16, unroll=_SC_UNROLL)
        def _(i):
            v = buf[pl.ds(i, 16)]
            bits = plsc.bitcast(v, jnp.int32)
            mag = jnp.bitwise_and(bits, jnp.int32(0x7FFFFFFF))
            pos = bits > 0
            if pass2:
                hi = jnp.right_shift(mag, 16)
                msk = jnp.logical_and(pos, hi == pv)
                idx = jnp.bitwise_and(mag, jnp.int32(0xFFFF))
            else:
                msk = pos
                idx = jnp.right_shift(mag, 16)
            plsc.addupdate_scatter(hist_v, [idx], ones16, mask=msk)

    def src(seg):
        return h_hbm.at[pl.ds(base + seg * _SC_CHUNK, _SC_CHUNK)]

    npair = nseg // 2
    pltpu.async_copy(src(0), buf_a, sem_a)

    def pair_body(p, carry):
        pltpu.async_copy(src(2 * p + 1), buf_b, sem_b)
        pltpu.make_async_copy(src(2 * p), buf_a, sem_a).wait()
        process(buf_a)

        @pl.when(p < npair - 1)
        def _():
            pltpu.async_copy(src(2 * p + 2), buf_a, sem_a)

        pltpu.make_async_copy(src(2 * p + 1), buf_b, sem_b).wait()
        process(buf_b)
        return carry

    lax.fori_loop(0, npair, pair_body, 0)
    pltpu.sync_copy(hist_v, out_hbm.at[wid])


def _sc_hist(flat, prefix, pass2, nbuck):
    n = flat.shape[0]
    nseg = n // (_NW * _SC_CHUNK)
    mesh = plsc.VectorSubcoreMesh(core_axis_name="c", subcore_axis_name="s")
    body = functools.partial(_sc_hist_body, pass2=pass2, nseg=nseg,
                             nbuck=nbuck)
    k = pl.kernel(
        body,
        out_type=jax.ShapeDtypeStruct((_NW, nbuck), jnp.int32),
        mesh=mesh,
        compiler_params=pltpu.CompilerParams(needs_layout_passes=False),
        scratch_types=[
            pltpu.VMEM((16,), jnp.int32),
            pltpu.VMEM((_SC_CHUNK,), jnp.float32),
            pltpu.VMEM((_SC_CHUNK,), jnp.float32),
            pltpu.VMEM((nbuck,), jnp.int32),
            pltpu.SemaphoreType.DMA,
            pltpu.SemaphoreType.DMA,
        ],
    )
    pref_arr = jnp.full((16,), prefix, jnp.int32)
    return k(flat, pref_arr)


def _select_threshold_sc(z, target):
    flat = z.reshape(-1)
    hist1 = _sc_hist(flat, jnp.int32(0), False, _NB1)
    h1 = jnp.sum(hist1, axis=0, dtype=jnp.int32)
    s1 = jnp.cumsum(h1[::-1], dtype=jnp.int32)[::-1]
    s1e = jnp.concatenate([s1, jnp.zeros((1,), jnp.int32)])
    p_star = jnp.clip(jnp.sum((s1 >= target).astype(jnp.int32)) - 1,
                      0, _NB1 - 1)
    above = s1e[p_star + 1]
    t2 = target - above

    hist2 = _sc_hist(flat, p_star, True, _NB2)
    h2 = jnp.sum(hist2, axis=0, dtype=jnp.int32)
    s2 = jnp.cumsum(h2[::-1], dtype=jnp.int32)[::-1]
    l_star = jnp.clip(jnp.sum((s2 >= t2).astype(jnp.int32)) - 1,
                      0, _NB2 - 1)
    cnt = above + s2[l_star]
    t_bits = jnp.bitwise_or(jnp.left_shift(p_star, 16), l_star)
    t = lax.bitcast_convert_type(t_bits, jnp.float32)
    return t, cnt


def _mm_body(t_ref, a_ref, b_ref, bias_ref, o_ref, *, masked, nk):
    k = pl.program_id(2)
    a = a_ref[...]
    if masked:
        a = jnp.where(a >= t_ref[0, 0], a, 0.0)
    d = jnp.dot(a, b_ref[...], preferred_element_type=jnp.float32,
                precision=_PREC)
    if nk == 1:
        o_ref[...] = d + bias_ref[...]
    else:
        @pl.when(k == 0)
        def _():
            o_ref[...] = d

        @pl.when((k > 0) & (k < nk - 1))
        def _():
            o_ref[...] += d

        @pl.when(k == nk - 1)
        def _():
            o_ref[...] = o_ref[...] + d + bias_ref[...]


def _matmul(a, b, bias, t, bm, bn, bk):
    M, K = a.shape
    _, N = b.shape
    nm, nn, nk = M // bm, N // bn, K // bk
    masked = t is not None
    tt = t.reshape(1, 1) if masked else jnp.zeros((1, 1), jnp.float32)
    body = functools.partial(_mm_body, masked=masked, nk=nk)
    return pl.pallas_call(
        body,
        grid=(nm, nn, nk),
        in_specs=[
            pl.BlockSpec(memory_space=pltpu.SMEM),
            pl.BlockSpec((bm, bk), lambda m, n, k: (m, k)),
            pl.BlockSpec((bk, bn), lambda m, n, k: (k, n)),
            pl.BlockSpec((1, bn), lambda m, n, k: (0, n)),
        ],
        out_specs=pl.BlockSpec((bm, bn), lambda m, n, k: (m, n)),
        out_shape=jax.ShapeDtypeStruct((M, N), jnp.float32),
    )(tt, a, b, bias.reshape(1, -1))


def _dec1_body(t_ref, a_ref, b_ref, bias_ref, x_ref, o_ref, l2_ref, acc_ref,
               *, nk):
    m, n, k = pl.program_id(0), pl.program_id(1), pl.program_id(2)

    @pl.when((m == 0) & (n == 0) & (k == 0))
    def _():
        l2_ref[...] = jnp.zeros_like(l2_ref)

    @pl.when(k == 0)
    def _():
        acc_ref[...] = jnp.zeros_like(acc_ref)

    a = jnp.where(a_ref[...] >= t_ref[0, 0], a_ref[...], 0.0)
    acc_ref[...] += jnp.dot(a, b_ref[...], preferred_element_type=jnp.float32,
                            precision=_PREC)

    @pl.when(k == nk - 1)
    def _():
        res = acc_ref[...] + bias_ref[...]
        o_ref[...] = res
        d = res - x_ref[...]
        s = jnp.sum(d * d)
        l2_ref[...] = l2_ref[...] + jnp.full((1, 128), s / 128.0, jnp.float32)


def _dec1_matmul(a, b, bias, t, x, bm, bn, bk):
    M, K = a.shape
    _, N = b.shape
    nm, nn, nk = M // bm, N // bn, K // bk
    return pl.pallas_call(
        functools.partial(_dec1_body, nk=nk),
        grid=(nm, nn, nk),
        in_specs=[
            pl.BlockSpec(memory_space=pltpu.SMEM),
            pl.BlockSpec((bm, bk), lambda m, n, k: (m, k)),
            pl.BlockSpec((bk, bn), lambda m, n, k: (k, n)),
            pl.BlockSpec((1, bn), lambda m, n, k: (0, n)),
            pl.BlockSpec((bm, bn), lambda m, n, k: (m, n)),
        ],
        out_specs=[
            pl.BlockSpec((bm, bn), lambda m, n, k: (m, n)),
            pl.BlockSpec((1, 128), lambda m, n, k: (0, 0)),
        ],
        out_shape=[
            jax.ShapeDtypeStruct((M, N), jnp.float32),
            jax.ShapeDtypeStruct((1, 128), jnp.float32),
        ],
        scratch_shapes=[pltpu.VMEM((bm, bn), jnp.float32)],
    )(t.reshape(1, 1), a, b, bias.reshape(1, -1), x)


def kernel(x, W_enc1, b_enc1, W_enc2, b_enc2, W_dec2, b_dec2, W_dec1, b_dec1):
    z1 = _matmul(x, W_enc1, b_enc1, None, bm=1024, bn=1024, bk=1024)
    t1, _ = _select_threshold_sc(z1, K_MID * B)

    z2 = _matmul(z1, W_enc2, b_enc2, t1, bm=2048, bn=1024, bk=1024)
    t2, cnt2 = _select_threshold_sc(z2, K_FEAT * B)

    z3 = _matmul(z2, W_dec2, b_dec2, t2, bm=2048, bn=1024, bk=1024)
    t3, _ = _select_threshold_sc(z3, K_MID * B)

    recon, l2part = _dec1_matmul(z3, W_dec1, b_dec1, t3, x,
                                 bm=1024, bn=1024, bk=1024)

    l2_loss = jnp.sum(l2part) / (B * D_MODEL)
    l0_norm = cnt2.astype(jnp.float32) / B
    n_dead = jnp.zeros((D_FEAT,), dtype=bool)
    return recon, l2_loss, l0_norm, n_dead
